# baseline (speedup 1.0000x reference)
import functools

import jax
import jax.numpy as jnp
from jax import lax
from jax.experimental import pallas as pl
from jax.experimental.pallas import tpu as pltpu
from jax.experimental.pallas import tpu_sc as plsc

BATCH = 20000
MEMORY_DIM = 256
GATES = 3 * MEMORY_DIM

CHUNK = 128
N_SUBCORES = 16
CHUNKS_FAST = 7
CHUNKS_SLOW = 3
FAST_CORE = 0
N_CHUNKS = N_SUBCORES * (CHUNKS_FAST + CHUNKS_SLOW)
B_PAD = N_CHUNKS * CHUNK
NBUF = 3


def _sc_gather(table, idx_fast, idx_slow):
    mesh = plsc.VectorSubcoreMesh(core_axis_name="c", subcore_axis_name="s")

    @functools.partial(
        pl.kernel,
        mesh=mesh,
        out_type=jax.ShapeDtypeStruct((B_PAD, MEMORY_DIM), jnp.float32),
        scratch_types=[
            pltpu.VMEM((CHUNKS_FAST, CHUNK), jnp.int32),
            pltpu.VMEM((CHUNK, MEMORY_DIM), jnp.float32),
            pltpu.VMEM((CHUNK, MEMORY_DIM), jnp.float32),
            pltpu.VMEM((CHUNK, MEMORY_DIM), jnp.float32),
            pltpu.SemaphoreType.DMA,
            pltpu.SemaphoreType.DMA,
            pltpu.SemaphoreType.DMA,
            pltpu.SemaphoreType.DMA,
            pltpu.SemaphoreType.DMA,
            pltpu.SemaphoreType.DMA,
        ],
    )
    def gather_kernel(
        table_hbm, idxf_hbm, idxs_hbm, out_hbm, idx_v, buf0, buf1, buf2,
        g0, g1, g2, w0, w1, w2,
    ):
        cid = lax.axis_index("c")
        sid = lax.axis_index("s")
        bufs = (buf0, buf1, buf2)
        gsems = (g0, g1, g2)
        wsems = (w0, w1, w2)

        def pipeline(n_chunks, chunk0):
            def out_slice(j):
                return out_hbm.at[pl.ds((chunk0 + j) * CHUNK, CHUNK)]

            for j in range(min(NBUF, n_chunks)):
                pltpu.async_copy(table_hbm.at[idx_v.at[j]], bufs[j], gsems[j])
            for j in range(n_chunks):
                b = j % NBUF
                pltpu.make_async_copy(
                    table_hbm.at[idx_v.at[j]], bufs[b], gsems[b]
                ).wait()
                pltpu.async_copy(bufs[b], out_slice(j), wsems[b])
                if j + NBUF < n_chunks:
                    pltpu.make_async_copy(bufs[b], out_slice(j), wsems[b]).wait()
                    pltpu.async_copy(
                        table_hbm.at[idx_v.at[j + NBUF]], bufs[b], gsems[b]
                    )
            for j in range(max(0, n_chunks - NBUF), n_chunks):
                b = j % NBUF
                pltpu.make_async_copy(bufs[b], out_slice(j), wsems[b]).wait()

        @pl.when(cid == FAST_CORE)
        def _fast():
            pltpu.sync_copy(idxf_hbm.at[sid], idx_v)
            pipeline(CHUNKS_FAST, sid * CHUNKS_FAST)

        @pl.when(cid != FAST_CORE)
        def _slow():
            pltpu.sync_copy(
                idxs_hbm.at[sid], idx_v.at[pl.ds(0, CHUNKS_SLOW)]
            )
            pipeline(CHUNKS_SLOW, N_SUBCORES * CHUNKS_FAST + sid * CHUNKS_SLOW)

    return gather_kernel(table, idx_fast, idx_slow)


def _tc_gru(mem_rows, w_hh_t, b_hh, bi_r, bi_z, bi_n):
    BM = 1000
    grid = (BATCH // BM,)

    def body(mem_ref, w_ref, bhh_ref, bir_ref, biz_ref, bin_ref, out_ref):
        h = mem_ref[...]
        gh = jnp.dot(h, w_ref[...], preferred_element_type=jnp.float32) + bhh_ref[...]
        h_r = gh[:, :MEMORY_DIM]
        h_z = gh[:, MEMORY_DIM : 2 * MEMORY_DIM]
        h_n = gh[:, 2 * MEMORY_DIM :]
        r = jax.nn.sigmoid(bir_ref[...] + h_r)
        z = jax.nn.sigmoid(biz_ref[...] + h_z)
        n = jnp.tanh(bin_ref[...] + r * h_n)
        out_ref[...] = (1.0 - z) * n + z * h

    return pl.pallas_call(
        body,
        grid=grid,
        in_specs=[
            pl.BlockSpec((BM, MEMORY_DIM), lambda i: (i, 0)),
            pl.BlockSpec((MEMORY_DIM, GATES), lambda i: (0, 0)),
            pl.BlockSpec((1, GATES), lambda i: (0, 0)),
            pl.BlockSpec((1, MEMORY_DIM), lambda i: (0, 0)),
            pl.BlockSpec((1, MEMORY_DIM), lambda i: (0, 0)),
            pl.BlockSpec((1, MEMORY_DIM), lambda i: (0, 0)),
        ],
        out_specs=pl.BlockSpec((BM, MEMORY_DIM), lambda i: (i, 0)),
        out_shape=jax.ShapeDtypeStruct((BATCH, MEMORY_DIM), jnp.float32),
        compiler_params=pltpu.CompilerParams(
            dimension_semantics=("parallel",),
        ),
    )(mem_rows, w_hh_t, b_hh, bi_r, bi_z, bi_n)


def kernel(n_id, memory, last_update, w_ih, w_hh, b_ih, b_hh):
    batch = n_id.shape[0]
    idx_flat = jnp.pad(n_id, (0, B_PAD - batch)).reshape(N_CHUNKS, CHUNK)
    n_fast = N_SUBCORES * CHUNKS_FAST
    idx_fast = idx_flat[:n_fast].reshape(N_SUBCORES, CHUNKS_FAST, CHUNK)
    idx_slow = idx_flat[n_fast:].reshape(N_SUBCORES, CHUNKS_SLOW, CHUNK)
    mem_rows = _sc_gather(memory, idx_fast, idx_slow)
    new_mem = _tc_gru(
        mem_rows,
        w_hh.T,
        b_hh.reshape(1, GATES),
        b_ih[:MEMORY_DIM].reshape(1, MEMORY_DIM),
        b_ih[MEMORY_DIM : 2 * MEMORY_DIM].reshape(1, MEMORY_DIM),
        b_ih[2 * MEMORY_DIM :].reshape(1, MEMORY_DIM),
    )
    new_last_update = jnp.zeros((batch,), dtype=jnp.int32)
    return new_mem, new_last_update
